# baseline (speedup 1.0000x reference)
import functools

import jax
import jax.numpy as jnp
from jax import lax
from jax.experimental import pallas as pl
from jax.experimental.pallas import tpu as pltpu
from jax.experimental.pallas import tpu_sc as plsc

_B = 4096
_N = 8192
_ROWS_PER_BLOCK = 256


def _argmin_body(dist_ref, out_ref):
    d = dist_ref[...]
    m = jnp.min(d, axis=1, keepdims=True)
    ii = lax.broadcasted_iota(jnp.int32, d.shape, 1)
    cand = jnp.where(d == m, ii, jnp.int32(_N))
    out_ref[0, 0, :] = jnp.min(cand, axis=1)


def _argmin_indices(distances):
    nb = _B // _ROWS_PER_BLOCK
    out = pl.pallas_call(
        _argmin_body,
        grid=(nb,),
        in_specs=[pl.BlockSpec((_ROWS_PER_BLOCK, _N), lambda i: (i, 0))],
        out_specs=pl.BlockSpec((1, 1, _ROWS_PER_BLOCK), lambda i: (i, 0, 0)),
        out_shape=jax.ShapeDtypeStruct((nb, 1, _ROWS_PER_BLOCK), jnp.int32),
    )(distances)
    return out.reshape(_B)


def _sc_gather(reg_vals, win_idx):
    info = plsc.get_sparse_core_info()
    nc, ns, lanes = info.num_cores, info.num_subcores, info.num_lanes
    nw = nc * ns
    bpw = _B // nw
    nch = bpw // lanes
    table = reg_vals.reshape(_B * _N)
    mesh = plsc.VectorSubcoreMesh(core_axis_name="c", subcore_axis_name="s")

    @functools.partial(
        pl.kernel,
        mesh=mesh,
        out_type=jax.ShapeDtypeStruct((_B,), jnp.float32),
        scratch_types=[
            pltpu.VMEM((bpw,), jnp.int32),
            pltpu.VMEM((bpw,), jnp.int32),
            pltpu.VMEM((bpw,), jnp.float32),
            pltpu.SemaphoreType.DMA,
        ],
    )
    def gather_kernel(tab_hbm, idx_hbm, out_hbm, idx_v, flat_v, o_v, sem):
        wid = lax.axis_index("s") * nc + lax.axis_index("c")
        base = wid * bpw
        pltpu.sync_copy(idx_hbm.at[pl.ds(base, bpw)], idx_v)
        for c in range(nch):
            j = idx_v[pl.ds(c * lanes, lanes)]
            i_vec = lax.iota(jnp.int32, lanes) + (base + c * lanes)
            flat_v[pl.ds(c * lanes, lanes)] = i_vec * _N + j
        pltpu.async_copy(tab_hbm.at[flat_v], o_v, sem).wait()
        pltpu.sync_copy(o_v, out_hbm.at[pl.ds(base, bpw)])

    return gather_kernel(table, win_idx)


def kernel(reg_vals, distances):
    win_idx = _argmin_indices(distances)
    return _sc_gather(reg_vals, win_idx)


# speedup vs baseline: 2.2891x; 2.2891x over previous
import functools

import jax
import jax.numpy as jnp
from jax import lax
from jax.experimental import pallas as pl
from jax.experimental.pallas import tpu as pltpu
from jax.experimental.pallas import tpu_sc as plsc

_B = 4096
_N = 8192
_ROWS_PER_BLOCK = 256


def _argmin_body(dist_ref, out_ref):
    d = dist_ref[...]
    m = jnp.min(d, axis=1, keepdims=True)
    ii = lax.broadcasted_iota(jnp.int32, d.shape, 1)
    cand = jnp.where(d == m, ii, jnp.int32(_N))
    out_ref[0, 0, :] = jnp.min(cand, axis=1)


def _argmin_indices(distances):
    nb = _B // _ROWS_PER_BLOCK
    out = pl.pallas_call(
        _argmin_body,
        grid=(nb,),
        in_specs=[pl.BlockSpec((_ROWS_PER_BLOCK, _N), lambda i: (i, 0))],
        out_specs=pl.BlockSpec((1, 1, _ROWS_PER_BLOCK), lambda i: (i, 0, 0)),
        out_shape=jax.ShapeDtypeStruct((nb, 1, _ROWS_PER_BLOCK), jnp.int32),
    )(distances)
    return out.reshape(_B)


def _sc_gather(reg_vals, win_idx):
    info = plsc.get_sparse_core_info()
    nc, ns, lanes = info.num_cores, info.num_subcores, info.num_lanes
    nw = nc * ns
    bpw = _B // nw
    nch = bpw // lanes
    mesh = plsc.VectorSubcoreMesh(core_axis_name="c", subcore_axis_name="s")

    @functools.partial(
        pl.kernel,
        mesh=mesh,
        out_type=jax.ShapeDtypeStruct((_B,), jnp.float32),
        compiler_params=pltpu.CompilerParams(needs_layout_passes=False),
        scratch_types=[
            pltpu.VMEM((bpw,), jnp.int32),
            pltpu.VMEM((bpw * 8,), jnp.float32),
            pltpu.VMEM((bpw,), jnp.float32),
            pltpu.SemaphoreType.DMA,
        ],
    )
    def gather_kernel(tab_hbm, idx_hbm, out_hbm, idx_v, buf_v, o_v, sem):
        wid = lax.axis_index("s") * nc + lax.axis_index("c")
        base = wid * bpw
        pltpu.sync_copy(idx_hbm.at[pl.ds(base, bpw)], idx_v)
        copies = []
        for c in range(nch):
            j16 = idx_v[pl.ds(c * lanes, lanes)]
            for m in range(lanes):
                k = c * lanes + m
                col0 = lax.div(j16[m], 8) * 8
                copies.append(pltpu.make_async_copy(
                    tab_hbm.at[base + k, pl.ds(col0, 8)],
                    buf_v.at[pl.ds(k * 8, 8)],
                    sem,
                ))
        for cp in copies:
            cp.start()
        for cp in copies:
            cp.wait()
        for c in range(nch):
            j = idx_v[pl.ds(c * lanes, lanes)]
            loc = (lax.iota(jnp.int32, lanes) + (c * lanes)) * 8 + lax.rem(
                j, 8)
            o_v[pl.ds(c * lanes, lanes)] = plsc.load_gather(buf_v, [loc])
        pltpu.sync_copy(o_v, out_hbm.at[pl.ds(base, bpw)])

    return gather_kernel(reg_vals, win_idx)


def kernel(reg_vals, distances):
    win_idx = _argmin_indices(distances)
    return _sc_gather(reg_vals, win_idx)


# speedup vs baseline: 2.3588x; 1.0304x over previous
import functools

import jax
import jax.numpy as jnp
from jax import lax
from jax.experimental import pallas as pl
from jax.experimental.pallas import tpu as pltpu
from jax.experimental.pallas import tpu_sc as plsc

_B = 4096
_N = 8192
_ROWS_PER_BLOCK = 512


def _argmin_body(dist_ref, out_ref):
    d = dist_ref[...]
    m = jnp.min(d, axis=1, keepdims=True)
    ii = lax.broadcasted_iota(jnp.int32, d.shape, 1)
    cand = jnp.where(d == m, ii, jnp.int32(_N))
    out_ref[0, 0, :] = jnp.min(cand, axis=1)


def _argmin_indices(distances):
    nb = _B // _ROWS_PER_BLOCK
    out = pl.pallas_call(
        _argmin_body,
        grid=(nb,),
        in_specs=[pl.BlockSpec((_ROWS_PER_BLOCK, _N), lambda i: (i, 0))],
        out_specs=pl.BlockSpec((1, 1, _ROWS_PER_BLOCK), lambda i: (i, 0, 0)),
        out_shape=jax.ShapeDtypeStruct((nb, 1, _ROWS_PER_BLOCK), jnp.int32),
    )(distances)
    return out.reshape(_B)


def _sc_gather(reg_vals, win_idx):
    info = plsc.get_sparse_core_info()
    nc, ns, lanes = info.num_cores, info.num_subcores, info.num_lanes
    nw = nc * ns
    bpw = _B // nw
    nch = bpw // lanes
    mesh = plsc.VectorSubcoreMesh(core_axis_name="c", subcore_axis_name="s")

    @functools.partial(
        pl.kernel,
        mesh=mesh,
        out_type=jax.ShapeDtypeStruct((_B,), jnp.float32),
        compiler_params=pltpu.CompilerParams(needs_layout_passes=False),
        scratch_types=[
            pltpu.VMEM((bpw,), jnp.int32),
            pltpu.VMEM((bpw * 8,), jnp.float32),
            pltpu.VMEM((bpw,), jnp.float32),
            pltpu.SemaphoreType.DMA,
        ],
    )
    def gather_kernel(tab_hbm, idx_hbm, out_hbm, idx_v, buf_v, o_v, sem):
        wid = lax.axis_index("s") * nc + lax.axis_index("c")
        base = wid * bpw
        pltpu.sync_copy(idx_hbm.at[pl.ds(base, bpw)], idx_v)
        copies = []
        for c in range(nch):
            j16 = idx_v[pl.ds(c * lanes, lanes)]
            for m in range(lanes):
                k = c * lanes + m
                col0 = lax.div(j16[m], 8) * 8
                copies.append(pltpu.make_async_copy(
                    tab_hbm.at[base + k, pl.ds(col0, 8)],
                    buf_v.at[pl.ds(k * 8, 8)],
                    sem,
                ))
        for cp in copies:
            cp.start()
        for cp in copies:
            cp.wait()
        for c in range(nch):
            j = idx_v[pl.ds(c * lanes, lanes)]
            loc = (lax.iota(jnp.int32, lanes) + (c * lanes)) * 8 + lax.rem(
                j, 8)
            o_v[pl.ds(c * lanes, lanes)] = plsc.load_gather(buf_v, [loc])
        pltpu.sync_copy(o_v, out_hbm.at[pl.ds(base, bpw)])

    return gather_kernel(reg_vals, win_idx)


def kernel(reg_vals, distances):
    win_idx = _argmin_indices(distances)
    return _sc_gather(reg_vals, win_idx)
